# baseline (speedup 1.0000x reference)
import functools

import jax
import jax.numpy as jnp
from jax import lax
from jax.experimental import pallas as pl
from jax.experimental.pallas import tpu as pltpu
from jax.experimental.pallas import tpu_sc as plsc

H = 128
N_AUTHOR = 1024
N_PAPER = 4096
LANES = 16
NC, NS = 2, 16
NW = NC * NS
P_PER_W = N_PAPER // NW
A_PER_W = N_AUTHOR // NW


def _sc_gather(paper_emb, author_emb, author_social_emb, venue_emb,
               ids_paper, ids_author, ids_venue):
    mesh = plsc.VectorSubcoreMesh(core_axis_name="c", subcore_axis_name="s")

    @functools.partial(
        pl.kernel,
        mesh=mesh,
        out_type=(
            jax.ShapeDtypeStruct((NW, H), jnp.float32),
            jax.ShapeDtypeStruct((N_AUTHOR, H), jnp.float32),
            jax.ShapeDtypeStruct((N_AUTHOR, H), jnp.float32),
            jax.ShapeDtypeStruct((1, H), jnp.float32),
        ),
        scratch_types=[
            pltpu.VMEM((P_PER_W,), jnp.int32),
            pltpu.VMEM((P_PER_W, H), jnp.float32),
            pltpu.VMEM((A_PER_W,), jnp.int32),
            pltpu.VMEM((A_PER_W, H), jnp.float32),
            pltpu.VMEM((A_PER_W, H), jnp.float32),
            pltpu.VMEM((1,), jnp.int32),
            pltpu.VMEM((1, H), jnp.float32),
            pltpu.VMEM((1, H), jnp.float32),
            pltpu.SemaphoreType.DMA,
            pltpu.SemaphoreType.DMA,
            pltpu.SemaphoreType.DMA,
            pltpu.SemaphoreType.DMA,
            pltpu.SemaphoreType.DMA,
        ],
    )
    def k(paper_hbm, aut_hbm, asoc_hbm, ven_hbm, idsp_hbm, idsa_hbm, idsv_hbm,
          psum_out, atop_out, asoc_out, ven_out,
          idx_p, rows_p, idx_a, rows_at, rows_as, idx_v, row_v, acc,
          sem_p, sem_p2, sem_at, sem_as, sem_v):
        wid = lax.axis_index("s") * NC + lax.axis_index("c")
        base_p = wid * P_PER_W
        base_a = wid * A_PER_W
        half = P_PER_W // 2
        nvec = H // LANES

        ci_a = pltpu.async_copy(idsa_hbm.at[pl.ds(base_a, A_PER_W)], idx_a, sem_at)
        ci_p = pltpu.async_copy(idsp_hbm.at[pl.ds(base_p, P_PER_W)], idx_p, sem_p)
        ci_a.wait()
        cp_at = pltpu.async_copy(aut_hbm.at[idx_a], rows_at, sem_at)
        cp_as = pltpu.async_copy(asoc_hbm.at[idx_a], rows_as, sem_as)
        ci_p.wait()
        cp_p = pltpu.async_copy(paper_hbm.at[idx_p], rows_p, sem_p)

        @pl.when(wid == 0)
        def _():
            pltpu.sync_copy(idsv_hbm, idx_v)
            pltpu.async_copy(ven_hbm.at[idx_v], row_v, sem_v).wait()
            pltpu.sync_copy(row_v, ven_out)

        cp_at.wait()
        co_at = pltpu.async_copy(rows_at, atop_out.at[pl.ds(base_a, A_PER_W)],
                                 sem_at)
        cp_as.wait()
        co_as = pltpu.async_copy(rows_as, asoc_out.at[pl.ds(base_a, A_PER_W)],
                                 sem_as)

        def body(r, carry):
            return tuple(carry[j] + rows_p[r, pl.ds(j * LANES, LANES)]
                         for j in range(nvec))

        zeros = tuple(jnp.zeros((LANES,), jnp.float32) for _ in range(nvec))
        cp_p.wait()
        sums = lax.fori_loop(0, P_PER_W, body, zeros, unroll=2)
        for j in range(nvec):
            acc[0, pl.ds(j * LANES, LANES)] = sums[j]
        pltpu.sync_copy(acc, psum_out.at[pl.ds(wid, 1)])
        co_at.wait()
        co_as.wait()

    return k(paper_emb, author_emb, author_social_emb, venue_emb,
             ids_paper, ids_author, ids_venue)


def _tc_body(atop, asoc, w1, b1, w2, b2, psum, ven, topic, wvec, out):
    f32 = jnp.float32
    hp = None
    at = atop[...]
    as_ = asoc[...]
    h = lax.dot_general(at, w1[0:H, :], (((1,), (0,)), ((), ())),
                        precision=hp, preferred_element_type=f32)
    h = h + lax.dot_general(as_, w1[H:2 * H, :], (((1,), (0,)), ((), ())),
                            precision=hp, preferred_element_type=f32)
    h = jnp.maximum(h + b1[...], 0.0)
    logits = lax.dot_general(h, w2[...], (((1,), (0,)), ((), ())),
                             precision=hp, preferred_element_type=f32)
    logits = logits + b2[...]
    m = jnp.max(logits)
    e = jnp.exp(logits - m)
    inv = 1.0 / jnp.sum(e)
    pa_t = lax.dot_general(e, at, (((0,), (0,)), ((), ())),
                           precision=hp, preferred_element_type=f32) * inv
    pa_s = lax.dot_general(e, as_, (((0,), (0,)), ((), ())),
                           precision=hp, preferred_element_type=f32) * inv
    paper = jnp.sum(psum[...], axis=0, keepdims=True) * (1.0 / N_PAPER)
    venue = ven[...]
    w = wvec[...]
    we = jnp.exp(w - jnp.max(w))
    ws = we / jnp.sum(we)
    w0 = ws[0:1, 0:1]
    w1s = ws[0:1, 1:2]
    w2s = ws[0:1, 2:3]
    w3s = ws[0:1, 3:4]
    lo = w0 * pa_t + w1s * venue + w2s * paper + w3s * topic[...]
    hi = w0 * pa_s
    out[0:1, 0:H] = lo
    out[0:1, H:2 * H] = hi


def _tc_call(atop, asoc, W1, b1, W2, b2, psum, ven, topic_vec, wvec,
             interpret=False):
    return pl.pallas_call(
        _tc_body,
        out_shape=jax.ShapeDtypeStruct((1, 2 * H), jnp.float32),
        interpret=interpret,
    )(atop, asoc, W1, b1.reshape(1, H // 2), W2, b2.reshape(1, 1),
      psum, ven, topic_vec.reshape(1, H), wvec)


def kernel(paper_emb, author_emb, author_social_emb, venue_emb, topic_vec,
           W1, b1, W2, b2, w_author, w_venue, w_paper, w_self,
           ids_author, ids_venue, ids_paper):
    psum, atop, asoc, ven = _sc_gather(
        paper_emb, author_emb, author_social_emb, venue_emb,
        ids_paper, ids_author, ids_venue)
    wvec = jnp.stack([w_author, w_venue, w_paper, w_self]).reshape(1, 4)
    out = _tc_call(atop, asoc, W1, b1, W2, b2, psum, ven, topic_vec, wvec)
    return out.reshape(2 * H)


# speedup vs baseline: 1.0019x; 1.0019x over previous
import functools

import jax
import jax.numpy as jnp
from jax import lax
from jax.experimental import pallas as pl
from jax.experimental.pallas import tpu as pltpu
from jax.experimental.pallas import tpu_sc as plsc

H = 128
N_AUTHOR = 1024
N_PAPER = 4096
LANES = 16
NC, NS = 2, 16
NW = NC * NS
P_PER_W = N_PAPER // NW
A_PER_W = N_AUTHOR // NW


def _sc_gather(paper_emb, author_emb, author_social_emb, venue_emb,
               ids_paper, ids_author, ids_venue):
    mesh = plsc.VectorSubcoreMesh(core_axis_name="c", subcore_axis_name="s")

    @functools.partial(
        pl.kernel,
        mesh=mesh,
        out_type=(
            jax.ShapeDtypeStruct((NW, H), jnp.float32),
            jax.ShapeDtypeStruct((N_AUTHOR, H), jnp.float32),
            jax.ShapeDtypeStruct((N_AUTHOR, H), jnp.float32),
            jax.ShapeDtypeStruct((1, H), jnp.float32),
        ),
        scratch_types=[
            pltpu.VMEM((P_PER_W,), jnp.int32),
            pltpu.VMEM((P_PER_W, H), jnp.float32),
            pltpu.VMEM((A_PER_W,), jnp.int32),
            pltpu.VMEM((A_PER_W, H), jnp.float32),
            pltpu.VMEM((A_PER_W, H), jnp.float32),
            pltpu.VMEM((1,), jnp.int32),
            pltpu.VMEM((1, H), jnp.float32),
            pltpu.VMEM((1, H), jnp.float32),
            pltpu.SemaphoreType.DMA,
            pltpu.SemaphoreType.DMA,
            pltpu.SemaphoreType.DMA,
            pltpu.SemaphoreType.DMA,
            pltpu.SemaphoreType.DMA,
        ],
    )
    def k(paper_hbm, aut_hbm, asoc_hbm, ven_hbm, idsp_hbm, idsa_hbm, idsv_hbm,
          psum_out, atop_out, asoc_out, ven_out,
          idx_p, rows_p, idx_a, rows_at, rows_as, idx_v, row_v, acc,
          sem_p, sem_p2, sem_at, sem_as, sem_v):
        wid = lax.axis_index("s") * NC + lax.axis_index("c")
        base_p = wid * P_PER_W
        base_a = wid * A_PER_W
        half = P_PER_W // 2
        nvec = H // LANES

        ci_a = pltpu.async_copy(idsa_hbm.at[pl.ds(base_a, A_PER_W)], idx_a, sem_at)
        ci_p = pltpu.async_copy(idsp_hbm.at[pl.ds(base_p, P_PER_W)], idx_p, sem_p)
        ci_a.wait()
        cp_at = pltpu.async_copy(aut_hbm.at[idx_a], rows_at, sem_at)
        cp_as = pltpu.async_copy(asoc_hbm.at[idx_a], rows_as, sem_as)
        ci_p.wait()
        cp_p = pltpu.async_copy(paper_hbm.at[idx_p], rows_p, sem_p)

        @pl.when(wid == 0)
        def _():
            pltpu.sync_copy(idsv_hbm, idx_v)
            pltpu.async_copy(ven_hbm.at[idx_v], row_v, sem_v).wait()
            pltpu.sync_copy(row_v, ven_out)

        cp_at.wait()
        co_at = pltpu.async_copy(rows_at, atop_out.at[pl.ds(base_a, A_PER_W)],
                                 sem_at)
        cp_as.wait()
        co_as = pltpu.async_copy(rows_as, asoc_out.at[pl.ds(base_a, A_PER_W)],
                                 sem_as)

        def body(r, carry):
            return tuple(carry[j] + rows_p[r, pl.ds(j * LANES, LANES)]
                         for j in range(nvec))

        zeros = tuple(jnp.zeros((LANES,), jnp.float32) for _ in range(nvec))
        cp_p.wait()
        sums = lax.fori_loop(0, P_PER_W, body, zeros, unroll=2)
        for j in range(nvec):
            acc[0, pl.ds(j * LANES, LANES)] = sums[j]
        pltpu.sync_copy(acc, psum_out.at[pl.ds(wid, 1)])
        co_at.wait()
        co_as.wait()

    return k(paper_emb, author_emb, author_social_emb, venue_emb,
             ids_paper, ids_author, ids_venue)


def _tc_body(atop, asoc, pack, w2, psum, ven, topic, out):
    f32 = jnp.float32
    hp = None
    at = atop[...]
    as_ = asoc[...]
    h = lax.dot_general(at, pack[0:H, :], (((1,), (0,)), ((), ())),
                        precision=hp, preferred_element_type=f32)
    h = h + lax.dot_general(as_, pack[H:2 * H, :], (((1,), (0,)), ((), ())),
                            precision=hp, preferred_element_type=f32)
    h = jnp.maximum(h + pack[2 * H:2 * H + 1, :], 0.0)
    logits = lax.dot_general(h, w2[...], (((1,), (0,)), ((), ())),
                             precision=hp, preferred_element_type=f32)
    logits = logits + jnp.sum(pack[2 * H + 1:2 * H + 2, :])
    m = jnp.max(logits)
    e = jnp.exp(logits - m)
    inv = 1.0 / jnp.sum(e)
    pa_t = lax.dot_general(e, at, (((0,), (0,)), ((), ())),
                           precision=hp, preferred_element_type=f32) * inv
    pa_s = lax.dot_general(e, as_, (((0,), (0,)), ((), ())),
                           precision=hp, preferred_element_type=f32) * inv
    paper = jnp.sum(psum[...], axis=0, keepdims=True) * (1.0 / N_PAPER)
    venue = ven[...]
    w = pack[2 * H + 2:2 * H + 3, 0:4]
    we = jnp.exp(w - jnp.max(w))
    ws = we / jnp.sum(we)
    w0 = ws[0:1, 0:1]
    w1s = ws[0:1, 1:2]
    w2s = ws[0:1, 2:3]
    w3s = ws[0:1, 3:4]
    lo = w0 * pa_t + w1s * venue + w2s * paper + w3s * topic[...]
    hi = w0 * pa_s
    out[0:1, 0:H] = lo
    out[0:1, H:2 * H] = hi


def _tc_call(atop, asoc, W1, b1, W2, b2, psum, ven, topic_vec, wvec,
             interpret=False):
    zpad1 = jnp.zeros((H // 2 - 1,), jnp.float32)
    zpad4 = jnp.zeros((H // 2 - 4,), jnp.float32)
    row_b2 = jnp.concatenate([b2.reshape(1), zpad1]).reshape(1, H // 2)
    row_w = jnp.concatenate([wvec.reshape(4), zpad4]).reshape(1, H // 2)
    pack = jnp.concatenate(
        [W1, b1.reshape(1, H // 2), row_b2, row_w], axis=0)
    return pl.pallas_call(
        _tc_body,
        out_shape=jax.ShapeDtypeStruct((1, 2 * H), jnp.float32),
        interpret=interpret,
    )(atop, asoc, pack, W2, psum, ven, topic_vec.reshape(1, H))


def kernel(paper_emb, author_emb, author_social_emb, venue_emb, topic_vec,
           W1, b1, W2, b2, w_author, w_venue, w_paper, w_self,
           ids_author, ids_venue, ids_paper):
    psum, atop, asoc, ven = _sc_gather(
        paper_emb, author_emb, author_social_emb, venue_emb,
        ids_paper, ids_author, ids_venue)
    wvec = jnp.stack([w_author, w_venue, w_paper, w_self]).reshape(1, 4)
    out = _tc_call(atop, asoc, W1, b1, W2, b2, psum, ven, topic_vec, wvec)
    return out.reshape(2 * H)
